# baseline (speedup 1.0000x reference)
import functools
import jax
import jax.numpy as jnp
from jax import lax
from jax.experimental import pallas as pl
from jax.experimental.pallas import tpu as pltpu
from jax.experimental.pallas import tpu_sc as plsc

B = 65536
HW = 361
OW = 3 * HW
NC, NS = 2, 16
NW = NC * NS
BPW = B // NW
NB = 32
NCHUNK = BPW // NB
NV = HW // 16

_mesh = plsc.VectorSubcoreMesh(core_axis_name="c", subcore_axis_name="s")


@functools.partial(
    pl.kernel,
    mesh=_mesh,
    out_type=jax.ShapeDtypeStruct((B * OW,), jnp.float32),
    scratch_types=[
        pltpu.VMEM((NB * HW,), jnp.int32),
        pltpu.VMEM((NB + 16,), jnp.int32),
        pltpu.VMEM((NB * OW,), jnp.float32),
    ],
    compiler_params=pltpu.CompilerParams(use_tc_tiling_on_sc=True),
)
def _sc_body(x_hbm, pls_hbm, out_hbm, xv, pv, ov):
    wid = lax.axis_index("s") * NC + lax.axis_index("c")
    board0 = wid * BPW
    one = jnp.ones((16,), jnp.float32)
    zero = jnp.zeros((16,), jnp.float32)
    two = jnp.full((16,), 2, jnp.int32)

    def chunk_body(ci, carry):
        b0 = board0 + ci * NB
        pltpu.sync_copy(x_hbm.at[pl.ds(b0 * HW, NB * HW)], xv)
        pltpu.sync_copy(pls_hbm.at[pl.ds(b0, NB)], pv.at[pl.ds(0, NB)])

        @plsc.parallel_loop(0, NB, unroll=1)
        def board_body(b):
            t0 = pv[pl.ds(b, 16)][0]
            tv = jnp.full((16,), t0, jnp.int32)
            uv = 1 - tv
            xoff = b * HW
            ooff = b * OW
            offs = tuple(j * 16 for j in range(NV)) + (HW - 16,)
            for c, cv in ((0, tv), (1, uv), (2, two)):
                for o in offs:
                    v = xv[pl.ds(xoff + o, 16)]
                    ov[pl.ds(ooff + c * HW + o, 16)] = jnp.where(v == cv, one, zero)

        pltpu.sync_copy(ov, out_hbm.at[pl.ds(b0 * OW, NB * OW)])
        return carry

    lax.fori_loop(0, NCHUNK, chunk_body, 0, unroll=False)


def kernel(x, pls):
    out = _sc_body(x.reshape(B * HW), pls)
    return out.reshape(B, 3, 19, 19)


# speedup vs baseline: 5.5642x; 5.5642x over previous
import jax
import jax.numpy as jnp
from jax.experimental import pallas as pl

B = 65536
HW = 361
BB = 512


def _body(x_ref, p_ref, o_ref):
    x = x_ref[...][:, None, :]
    t0 = p_ref[...][:, :, None]
    ci = jax.lax.broadcasted_iota(jnp.int32, (1, 3, 1), 1)
    tgt = jnp.where(ci == 0, t0, jnp.where(ci == 1, 1 - t0, jnp.full_like(t0, 2)))
    o_ref[...] = (x == tgt).astype(jnp.float32)


def kernel(x, pls):
    xf = x.reshape(B, HW)
    pf = pls.reshape(B, 1)
    out = pl.pallas_call(
        _body,
        grid=(B // BB,),
        in_specs=[
            pl.BlockSpec((BB, HW), lambda i: (i, 0)),
            pl.BlockSpec((BB, 1), lambda i: (i, 0)),
        ],
        out_specs=pl.BlockSpec((BB, 3, HW), lambda i: (i, 0, 0)),
        out_shape=jax.ShapeDtypeStruct((B, 3, HW), jnp.float32),
    )(xf, pf)
    return out.reshape(B, 3, 19, 19)
